# Initial kernel scaffold; baseline (speedup 1.0000x reference)
import jax
import jax.numpy as jnp
from jax.experimental import pallas as pl


def kernel(x_src, x_dst, time_src, seed_time, W_enc_src, b_enc_src, W_enc_dst, b_enc_dst, W_time, b_time, emb_shallow, id_emb, W_root_s0, W_nbr_s0, W_root_d0, W_nbr_d0, W_root_s1, W_nbr_s1, W_root_d1, W_nbr_d1, ln_gamma, ln_beta, W_head, b_head, batch_src, n_id_dst, edge_index):
    raise NotImplementedError("write your pallas kernel here")


# speedup vs baseline: 3.9511x; 3.9511x over previous
import functools

import jax
import jax.numpy as jnp
from jax import lax
from jax.experimental import pallas as pl
from jax.experimental.pallas import tpu as pltpu
from jax.experimental.pallas import tpu_sc as plsc

_SC_PARAMS = pltpu.CompilerParams(use_tc_tiling_on_sc=False)

NC = 2
NS = 16
NW = NC * NS
LANE = 16
CHUNK = 80

_F32 = jnp.float32
_HIGH = lax.Precision.HIGHEST


def _fill2d(ref, rows, cols, value):
    vec = jnp.full((LANE,), value, _F32)

    def row_body(i, _):
        def col_body(j, __):
            ref[i, pl.ds(j * LANE, LANE)] = vec
            return 0
        return lax.fori_loop(0, cols // LANE, col_body, 0)

    lax.fori_loop(0, rows, row_body, 0)


def _padded_rows(n):
    return 128 * -(-n // (NS * 128))


@functools.lru_cache(maxsize=None)
def _make_sc_pre(vocab, c, nid_rows, e_rows, n_nodes):
    emb_rows_w = nid_rows // NW
    er_w = e_rows // NW
    zt = _padded_rows(n_nodes)
    npad = NS * zt
    mesh = plsc.VectorSubcoreMesh(core_axis_name="c", subcore_axis_name="s")

    @functools.partial(
        pl.kernel,
        out_type=[
            jax.ShapeDtypeStruct((nid_rows * 128, c), _F32),
            jax.ShapeDtypeStruct((NC, NS, zt, LANE), _F32),
            jax.ShapeDtypeStruct((NC, NS, zt, LANE), _F32),
        ],
        mesh=mesh,
        compiler_params=_SC_PARAMS,
        scratch_types=[
            pltpu.VMEM((emb_rows_w, 128), jnp.int32),
            pltpu.VMEM((128, c), _F32),
            pltpu.VMEM((er_w, CHUNK), jnp.int32),
            pltpu.VMEM((er_w, CHUNK), jnp.int32),
            pltpu.VMEM((CHUNK, LANE), _F32),
            pltpu.VMEM((zt, LANE), _F32),
            pltpu.VMEM_SHARED((npad, LANE), _F32),
            pltpu.VMEM_SHARED((npad, LANE), _F32),
            pltpu.SemaphoreType.DMA,
        ],
    )
    def pre(emb_hbm, nid_hbm, src_hbm, dst_hbm,
            emb_out, cntd_out, cnts_out,
            nid_v, rows_v, src_v, dst_v, ones_v, zero_v, accd, accs, sem):
        cid = lax.axis_index("c")
        sid = lax.axis_index("s")
        w = cid * NS + sid

        pltpu.sync_copy(nid_hbm.at[w], nid_v)
        for r in range(emb_rows_w):
            pltpu.async_copy(emb_hbm.at[nid_v.at[r]], rows_v, sem).wait()
            pltpu.sync_copy(
                rows_v, emb_out.at[pl.ds((w * emb_rows_w + r) * 128, 128), :])

        _fill2d(ones_v, CHUNK, LANE, 1.0)
        _fill2d(zero_v, zt, LANE, 0.0)
        pltpu.sync_copy(src_hbm.at[w], src_v)
        pltpu.sync_copy(dst_hbm.at[w], dst_v)
        pltpu.sync_copy(zero_v, accd.at[pl.ds(sid * zt, zt), :])
        pltpu.sync_copy(zero_v, accs.at[pl.ds(sid * zt, zt), :])
        plsc.subcore_barrier()

        def cnt_body(i, _):
            pltpu.sync_copy(ones_v, accd.at[dst_v.at[i]], add=True)
            pltpu.sync_copy(ones_v, accs.at[src_v.at[i]], add=True)
            return 0

        lax.fori_loop(0, er_w, cnt_body, 0)
        plsc.subcore_barrier()
        pltpu.sync_copy(accd.at[pl.ds(sid * zt, zt), :], cntd_out.at[cid, sid])
        pltpu.sync_copy(accs.at[pl.ds(sid * zt, zt), :], cnts_out.at[cid, sid])

    return pre


@functools.lru_cache(maxsize=None)
def _make_sc_agg(n_nodes, c, e_rows, ndir):
    ch = c // NC
    er_w = e_rows // NS
    zt = _padded_rows(n_nodes)
    nz = zt // 128
    npad = NS * zt
    mesh = plsc.VectorSubcoreMesh(core_axis_name="c", subcore_axis_name="s")

    @functools.partial(
        pl.kernel,
        out_type=[jax.ShapeDtypeStruct((NC, NS * nz, 128, ch), _F32)
                  for _ in range(ndir)],
        mesh=mesh,
        compiler_params=_SC_PARAMS,
        scratch_types=[
            pltpu.VMEM((er_w, CHUNK), jnp.int32),
            pltpu.VMEM((er_w, CHUNK), jnp.int32),
            pltpu.VMEM((CHUNK, ch), _F32),
            pltpu.VMEM((128, ch), _F32),
            pltpu.VMEM_SHARED((npad, ch), _F32),
            pltpu.SemaphoreType.DMA,
        ],
    )
    def agg(*refs):
        h_hbm = refs[0:ndir]
        src_hbm, dst_hbm = refs[ndir], refs[ndir + 1]
        outs = refs[ndir + 2: 2 * ndir + 2]
        src_v, dst_v, rows_v, zero_v, acc, sem = refs[2 * ndir + 2:]

        cid = lax.axis_index("c")
        sid = lax.axis_index("s")
        _fill2d(zero_v, 128, ch, 0.0)
        pltpu.sync_copy(src_hbm.at[sid], src_v)
        pltpu.sync_copy(dst_hbm.at[sid], dst_v)

        for d in range(ndir):
            gid_v = src_v if d == 0 else dst_v
            sid_v = dst_v if d == 0 else src_v
            for k in range(nz):
                pltpu.sync_copy(
                    zero_v, acc.at[pl.ds(sid * zt + k * 128, 128), :])
            plsc.subcore_barrier()

            def edge_body(i, _, h=h_hbm[d], g=gid_v, s=sid_v):
                pltpu.async_copy(h.at[cid].at[g.at[i]], rows_v, sem).wait()
                pltpu.sync_copy(rows_v, acc.at[s.at[i]], add=True)
                return 0

            lax.fori_loop(0, er_w, edge_body, 0)
            plsc.subcore_barrier()
            for k in range(nz):
                sl = pl.ds(sid * zt + k * 128, 128)
                pltpu.sync_copy(acc.at[sl, :], outs[d].at[cid, sid * nz + k])
            if d + 1 < ndir:
                plsc.subcore_barrier()

    return agg


def _dot(a, b):
    return jnp.dot(a, b, preferred_element_type=_F32, precision=_HIGH)


def _enc_body(nrows, blk, bsz,
              xs_ref, xd_ref, ts_ref, st_ref, bsrc_ref, emb_ref,
              cntd_ref, cnts_ref, wes_ref, bes_ref, wed_ref, bed_ref,
              wt_ref, bt_ref, ide_ref,
              hs_out, hd_out, invd_out, invs_out):
    i = pl.program_id(0)
    hs = _dot(xs_ref[...], wes_ref[...]) + bes_ref[0][None, :]
    bs = bsrc_ref[...]
    cols = lax.broadcasted_iota(jnp.int32, (blk, bsz), 1)
    stg = jnp.sum(jnp.where(cols == bs, st_ref[...], 0.0), axis=1,
                  keepdims=True)
    rel = stg - ts_ref[...]
    hs = hs + rel * wt_ref[0][None, :] + bt_ref[0][None, :]
    row = i * blk + lax.broadcasted_iota(jnp.int32, (blk, 1), 0)
    hs = hs + jnp.where(row < bsz, 1.0, 0.0) * ide_ref[0][None, :]
    hs_out[...] = hs
    hd_out[...] = (_dot(xd_ref[...], wed_ref[...]) + bed_ref[0][None, :]
                   + emb_ref[...])
    cd = cntd_ref[0, :, 0] + cntd_ref[1, :, 0]
    cs = cnts_ref[0, :, 0] + cnts_ref[1, :, 0]
    invd_out[...] = (1.0 / jnp.maximum(cd, 1.0))[:, None]
    invs_out[...] = (1.0 / jnp.maximum(cs, 1.0))[:, None]


def _upd_body(hd_ref, hs_ref, psd_ref, pds_ref, invd_ref, invs_ref,
              wrd_ref, wnd_ref, wrs_ref, wns_ref, hd_out, hs_out):
    m_sd = jnp.concatenate([psd_ref[0], psd_ref[1]], axis=-1) * invd_ref[...]
    m_ds = jnp.concatenate([pds_ref[0], pds_ref[1]], axis=-1) * invs_ref[...]
    hd_out[...] = jnp.maximum(
        _dot(hd_ref[...], wrd_ref[...]) + _dot(m_sd, wnd_ref[...]), 0.0)
    hs_out[...] = jnp.maximum(
        _dot(hs_ref[...], wrs_ref[...]) + _dot(m_ds, wns_ref[...]), 0.0)


def _fin_body(hd_ref, psd_ref, invd_ref, wrd_ref, wnd_ref,
              gam_ref, bet_ref, wh_ref, bh_ref, out_ref):
    m_sd = jnp.concatenate([psd_ref[0], psd_ref[1]], axis=-1) * invd_ref[...]
    h2 = jnp.maximum(
        _dot(hd_ref[...], wrd_ref[...]) + _dot(m_sd, wnd_ref[...]), 0.0)
    mu = jnp.mean(h2, axis=-1, keepdims=True)
    var = jnp.mean((h2 - mu) ** 2, axis=-1, keepdims=True)
    hn = (h2 - mu) / jnp.sqrt(var + 1e-5) * gam_ref[0][None, :] \
        + bet_ref[0][None, :]
    out_ref[...] = jnp.sum(hn * wh_ref[0][None, :], axis=1,
                           keepdims=True) + bh_ref[0, 0]


def _row_spec(blk, width):
    return pl.BlockSpec((blk, width), lambda i: (i, 0))


def _full_spec(shape):
    return pl.BlockSpec(shape, lambda i: tuple(0 for _ in shape))


def kernel(x_src, x_dst, time_src, seed_time, W_enc_src, b_enc_src,
           W_enc_dst, b_enc_dst, W_time, b_time, emb_shallow, id_emb,
           W_root_s0, W_nbr_s0, W_root_d0, W_nbr_d0, W_root_s1, W_nbr_s1,
           W_root_d1, W_nbr_d1, ln_gamma, ln_beta, W_head, b_head,
           batch_src, n_id_dst, edge_index):
    n_src, d_in = x_src.shape
    n_dst = x_dst.shape[0]
    c = W_enc_src.shape[1]
    e = edge_index.shape[1]
    bsz = seed_time.shape[0]
    vocab = emb_shallow.shape[0]
    assert e % (NW * CHUNK) == 0 and n_dst % (NS * 5) == 0 and n_src % (NS * 5) == 0

    e_rows = e // CHUNK
    src_idx = edge_index[0].reshape(NW, e_rows // NW, CHUNK)
    dst_idx = edge_index[1].reshape(NW, e_rows // NW, CHUNK)
    src_idx16 = edge_index[0].reshape(NS, e_rows // NS, CHUNK)
    dst_idx16 = edge_index[1].reshape(NS, e_rows // NS, CHUNK)
    ch = c // NC

    def _split(h):
        return jnp.stack([h[:, :ch], h[:, ch:]])

    nid_pad = NW * 128 * -(-n_dst // (NW * 128))
    nid3d = jnp.concatenate(
        [n_id_dst, jnp.zeros((nid_pad - n_dst,), jnp.int32)]
    ).reshape(NW, -1, 128)

    assert n_src == n_dst
    sc_pre = _make_sc_pre(vocab, c, nid_pad // 128, e_rows, n_dst)
    emb_rows, cnt_d, cnt_s = sc_pre(emb_shallow, nid3d, src_idx, dst_idx)
    emb_rows = emb_rows[:n_dst]
    cnt_d = cnt_d.reshape(NC, -1, LANE)[:, :n_dst]
    cnt_s = cnt_s.reshape(NC, -1, LANE)[:, :n_src]

    blk = 400
    grid = (n_src // blk,)
    row = functools.partial(_row_spec, blk)
    full = _full_spec
    cnt_spec = pl.BlockSpec((NC, blk, LANE), lambda i: (0, i, 0))
    part_spec = pl.BlockSpec((NC, blk, ch), lambda i: (0, i, 0))

    h_src, h_dst, inv_d, inv_s = pl.pallas_call(
        functools.partial(_enc_body, n_src, blk, bsz),
        grid=grid,
        in_specs=[row(d_in), row(d_in), row(1), full((1, bsz)), row(1),
                  row(c), cnt_spec, cnt_spec, full((d_in, c)), full((1, c)),
                  full((d_in, c)), full((1, c)), full((1, c)), full((1, c)),
                  full((1, c))],
        out_specs=[row(c), row(c), row(1), row(1)],
        out_shape=[jax.ShapeDtypeStruct((n_src, c), _F32),
                   jax.ShapeDtypeStruct((n_dst, c), _F32),
                   jax.ShapeDtypeStruct((n_dst, 1), _F32),
                   jax.ShapeDtypeStruct((n_src, 1), _F32)],
    )(x_src, x_dst, time_src[:, None], seed_time[None, :],
      batch_src[:, None], emb_rows, cnt_d, cnt_s,
      W_enc_src, b_enc_src[None, :], W_enc_dst, b_enc_dst[None, :],
      W_time, b_time[None, :], id_emb)

    agg2 = _make_sc_agg(n_dst, c, e_rows, 2)
    p_sd0, p_ds0 = agg2(_split(h_src), _split(h_dst), src_idx16, dst_idx16)
    p_sd0 = p_sd0.reshape(NC, -1, ch)[:, :n_dst]
    p_ds0 = p_ds0.reshape(NC, -1, ch)[:, :n_src]

    h_dst1, h_src1 = pl.pallas_call(
        _upd_body,
        grid=grid,
        in_specs=[row(c), row(c), part_spec, part_spec, row(1), row(1),
                  full((c, c)), full((c, c)), full((c, c)), full((c, c))],
        out_specs=[row(c), row(c)],
        out_shape=[jax.ShapeDtypeStruct((n_dst, c), _F32),
                   jax.ShapeDtypeStruct((n_src, c), _F32)],
    )(h_dst, h_src, p_sd0, p_ds0, inv_d, inv_s,
      W_root_d0, W_nbr_d0, W_root_s0, W_nbr_s0)

    agg1 = _make_sc_agg(n_dst, c, e_rows, 1)
    (p_sd1,) = agg1(_split(h_src1), src_idx16, dst_idx16)
    p_sd1 = p_sd1.reshape(NC, -1, ch)[:, :n_dst]

    out = pl.pallas_call(
        _fin_body,
        grid=grid,
        in_specs=[row(c), part_spec, row(1), full((c, c)), full((c, c)),
                  full((1, c)), full((1, c)), full((1, c)), full((1, 1))],
        out_specs=row(1),
        out_shape=jax.ShapeDtypeStruct((n_dst, 1), _F32),
    )(h_dst1, p_sd1, inv_d, W_root_d1, W_nbr_d1,
      ln_gamma[None, :], ln_beta[None, :], W_head.T, b_head[:, None])

    return out.reshape(-1)


# speedup vs baseline: 5.7304x; 1.4503x over previous
import functools

import jax
import jax.numpy as jnp
from jax import lax
from jax.experimental import pallas as pl
from jax.experimental.pallas import tpu as pltpu
from jax.experimental.pallas import tpu_sc as plsc

_SC_PARAMS = pltpu.CompilerParams(use_tc_tiling_on_sc=False)

NC = 2
NS = 16
NW = NC * NS
LANE = 16
CHUNK = 80

_F32 = jnp.float32
_HIGH = lax.Precision.HIGHEST


def _fill2d(ref, rows, cols, value):
    vec = jnp.full((LANE,), value, _F32)

    def row_body(i, _):
        def col_body(j, __):
            ref[i, pl.ds(j * LANE, LANE)] = vec
            return 0
        return lax.fori_loop(0, cols // LANE, col_body, 0)

    lax.fori_loop(0, rows, row_body, 0)


def _padded_rows(n):
    return 128 * -(-n // (NS * 128))


@functools.lru_cache(maxsize=None)
def _make_sc_pre(vocab, c, nid_rows, e_rows, n_nodes):
    emb_rows_w = nid_rows // NW
    er_w = e_rows // NW
    zt = _padded_rows(n_nodes)
    npad = NS * zt
    mesh = plsc.VectorSubcoreMesh(core_axis_name="c", subcore_axis_name="s")

    @functools.partial(
        pl.kernel,
        out_type=[
            jax.ShapeDtypeStruct((nid_rows * 128, c), _F32),
            jax.ShapeDtypeStruct((NC, NS, zt, LANE), _F32),
            jax.ShapeDtypeStruct((NC, NS, zt, LANE), _F32),
        ],
        mesh=mesh,
        compiler_params=_SC_PARAMS,
        scratch_types=[
            pltpu.VMEM((emb_rows_w, 128), jnp.int32),
            pltpu.VMEM((128, c), _F32),
            pltpu.VMEM((er_w, CHUNK), jnp.int32),
            pltpu.VMEM((er_w, CHUNK), jnp.int32),
            pltpu.VMEM((CHUNK, LANE), _F32),
            pltpu.VMEM((zt, LANE), _F32),
            pltpu.VMEM_SHARED((npad, LANE), _F32),
            pltpu.VMEM_SHARED((npad, LANE), _F32),
            pltpu.SemaphoreType.DMA,
        ],
    )
    def pre(emb_hbm, nid_hbm, src_hbm, dst_hbm,
            emb_out, cntd_out, cnts_out,
            nid_v, rows_v, src_v, dst_v, ones_v, zero_v, accd, accs, sem):
        cid = lax.axis_index("c")
        sid = lax.axis_index("s")
        w = cid * NS + sid

        pltpu.sync_copy(nid_hbm.at[w], nid_v)
        for r in range(emb_rows_w):
            pltpu.async_copy(emb_hbm.at[nid_v.at[r]], rows_v, sem).wait()
            pltpu.sync_copy(
                rows_v, emb_out.at[pl.ds((w * emb_rows_w + r) * 128, 128), :])

        _fill2d(ones_v, CHUNK, LANE, 1.0)
        _fill2d(zero_v, zt, LANE, 0.0)
        pltpu.sync_copy(src_hbm.at[w], src_v)
        pltpu.sync_copy(dst_hbm.at[w], dst_v)
        pltpu.sync_copy(zero_v, accd.at[pl.ds(sid * zt, zt), :])
        pltpu.sync_copy(zero_v, accs.at[pl.ds(sid * zt, zt), :])
        plsc.subcore_barrier()

        def cnt_body(i, _):
            pltpu.sync_copy(ones_v, accd.at[dst_v.at[i]], add=True)
            pltpu.sync_copy(ones_v, accs.at[src_v.at[i]], add=True)
            return 0

        lax.fori_loop(0, er_w, cnt_body, 0)
        plsc.subcore_barrier()
        pltpu.sync_copy(accd.at[pl.ds(sid * zt, zt), :], cntd_out.at[cid, sid])
        pltpu.sync_copy(accs.at[pl.ds(sid * zt, zt), :], cnts_out.at[cid, sid])

    return pre


@functools.lru_cache(maxsize=None)
def _make_sc_agg(n_nodes, c, e_rows, ndir):
    ch = c // NC
    er_w = e_rows // NS
    zt = _padded_rows(n_nodes)
    nz = zt // 128
    npad = NS * zt
    mesh = plsc.VectorSubcoreMesh(core_axis_name="c", subcore_axis_name="s")

    @functools.partial(
        pl.kernel,
        out_type=[jax.ShapeDtypeStruct((NC, NS * nz, 128, ch), _F32)
                  for _ in range(ndir)],
        mesh=mesh,
        compiler_params=_SC_PARAMS,
        scratch_types=[
            pltpu.VMEM((er_w, CHUNK), jnp.int32),
            pltpu.VMEM((er_w, CHUNK), jnp.int32),
            pltpu.VMEM((CHUNK, ch), _F32),
            pltpu.VMEM((CHUNK, ch), _F32),
            pltpu.VMEM((128, ch), _F32),
            pltpu.VMEM_SHARED((npad, ch), _F32),
            pltpu.SemaphoreType.DMA,
            pltpu.SemaphoreType.DMA,
        ],
    )
    def agg(*refs):
        h_hbm = refs[0:ndir]
        src_hbm, dst_hbm = refs[ndir], refs[ndir + 1]
        outs = refs[ndir + 2: 2 * ndir + 2]
        src_v, dst_v, rows0, rows1, zero_v, acc, sem0, sem1 = \
            refs[2 * ndir + 2:]

        cid = lax.axis_index("c")
        sid = lax.axis_index("s")
        _fill2d(zero_v, 128, ch, 0.0)
        pltpu.sync_copy(src_hbm.at[sid], src_v)
        pltpu.sync_copy(dst_hbm.at[sid], dst_v)

        for d in range(ndir):
            gid_v = src_v if d == 0 else dst_v
            sid_v = dst_v if d == 0 else src_v
            for k in range(nz):
                pltpu.sync_copy(
                    zero_v, acc.at[pl.ds(sid * zt + k * 128, 128), :])
            plsc.subcore_barrier()

            h = h_hbm[d].at[cid]
            g, s = gid_v, sid_v

            def _wait(buf, sem, h=h, g=g):
                pltpu.make_async_copy(h.at[g.at[0]], buf, sem).wait()

            pltpu.async_copy(h.at[g.at[0]], rows0, sem0)
            pltpu.async_copy(h.at[g.at[1]], rows1, sem1)

            def pair_body(p, _, h=h, g=g, s=s):
                j = 2 * p
                _wait(rows0, sem0)
                pltpu.sync_copy(rows0, acc.at[s.at[j]], add=True)
                pltpu.async_copy(h.at[g.at[j + 2]], rows0, sem0)
                _wait(rows1, sem1)
                pltpu.sync_copy(rows1, acc.at[s.at[j + 1]], add=True)
                pltpu.async_copy(h.at[g.at[j + 3]], rows1, sem1)
                return 0

            lax.fori_loop(0, er_w // 2 - 1, pair_body, 0)
            _wait(rows0, sem0)
            pltpu.sync_copy(rows0, acc.at[s.at[er_w - 2]], add=True)
            _wait(rows1, sem1)
            pltpu.sync_copy(rows1, acc.at[s.at[er_w - 1]], add=True)
            plsc.subcore_barrier()
            for k in range(nz):
                sl = pl.ds(sid * zt + k * 128, 128)
                pltpu.sync_copy(acc.at[sl, :], outs[d].at[cid, sid * nz + k])
            if d + 1 < ndir:
                plsc.subcore_barrier()

    return agg


def _dot(a, b):
    return jnp.dot(a, b, preferred_element_type=_F32, precision=_HIGH)


def _enc_body(nrows, blk, bsz,
              xs_ref, xd_ref, ts_ref, st_ref, bsrc_ref, emb_ref,
              cntd_ref, cnts_ref, wes_ref, bes_ref, wed_ref, bed_ref,
              wt_ref, bt_ref, ide_ref,
              hs_out, hd_out, invd_out, invs_out):
    i = pl.program_id(0)
    hs = _dot(xs_ref[...], wes_ref[...]) + bes_ref[0][None, :]
    bs = bsrc_ref[...]
    cols = lax.broadcasted_iota(jnp.int32, (blk, bsz), 1)
    stg = jnp.sum(jnp.where(cols == bs, st_ref[...], 0.0), axis=1,
                  keepdims=True)
    rel = stg - ts_ref[...]
    hs = hs + rel * wt_ref[0][None, :] + bt_ref[0][None, :]
    row = i * blk + lax.broadcasted_iota(jnp.int32, (blk, 1), 0)
    hs = hs + jnp.where(row < bsz, 1.0, 0.0) * ide_ref[0][None, :]
    hs_out[...] = hs
    hd_out[...] = (_dot(xd_ref[...], wed_ref[...]) + bed_ref[0][None, :]
                   + emb_ref[...])
    cd = cntd_ref[0, :, 0] + cntd_ref[1, :, 0]
    cs = cnts_ref[0, :, 0] + cnts_ref[1, :, 0]
    invd_out[...] = (1.0 / jnp.maximum(cd, 1.0))[:, None]
    invs_out[...] = (1.0 / jnp.maximum(cs, 1.0))[:, None]


def _upd_body(hd_ref, hs_ref, psd_ref, pds_ref, invd_ref, invs_ref,
              wrd_ref, wnd_ref, wrs_ref, wns_ref, hd_out, hs_out):
    m_sd = jnp.concatenate([psd_ref[0], psd_ref[1]], axis=-1) * invd_ref[...]
    m_ds = jnp.concatenate([pds_ref[0], pds_ref[1]], axis=-1) * invs_ref[...]
    hd_out[...] = jnp.maximum(
        _dot(hd_ref[...], wrd_ref[...]) + _dot(m_sd, wnd_ref[...]), 0.0)
    hs_out[...] = jnp.maximum(
        _dot(hs_ref[...], wrs_ref[...]) + _dot(m_ds, wns_ref[...]), 0.0)


def _fin_body(hd_ref, psd_ref, invd_ref, wrd_ref, wnd_ref,
              gam_ref, bet_ref, wh_ref, bh_ref, out_ref):
    m_sd = jnp.concatenate([psd_ref[0], psd_ref[1]], axis=-1) * invd_ref[...]
    h2 = jnp.maximum(
        _dot(hd_ref[...], wrd_ref[...]) + _dot(m_sd, wnd_ref[...]), 0.0)
    mu = jnp.mean(h2, axis=-1, keepdims=True)
    var = jnp.mean((h2 - mu) ** 2, axis=-1, keepdims=True)
    hn = (h2 - mu) / jnp.sqrt(var + 1e-5) * gam_ref[0][None, :] \
        + bet_ref[0][None, :]
    out_ref[...] = jnp.sum(hn * wh_ref[0][None, :], axis=1,
                           keepdims=True) + bh_ref[0, 0]


def _row_spec(blk, width):
    return pl.BlockSpec((blk, width), lambda i: (i, 0))


def _full_spec(shape):
    return pl.BlockSpec(shape, lambda i: tuple(0 for _ in shape))


def kernel(x_src, x_dst, time_src, seed_time, W_enc_src, b_enc_src,
           W_enc_dst, b_enc_dst, W_time, b_time, emb_shallow, id_emb,
           W_root_s0, W_nbr_s0, W_root_d0, W_nbr_d0, W_root_s1, W_nbr_s1,
           W_root_d1, W_nbr_d1, ln_gamma, ln_beta, W_head, b_head,
           batch_src, n_id_dst, edge_index):
    n_src, d_in = x_src.shape
    n_dst = x_dst.shape[0]
    c = W_enc_src.shape[1]
    e = edge_index.shape[1]
    bsz = seed_time.shape[0]
    vocab = emb_shallow.shape[0]
    assert e % (NW * CHUNK) == 0 and n_dst % (NS * 5) == 0 and n_src % (NS * 5) == 0

    e_rows = e // CHUNK
    src_idx = edge_index[0].reshape(NW, e_rows // NW, CHUNK)
    dst_idx = edge_index[1].reshape(NW, e_rows // NW, CHUNK)
    src_idx16 = edge_index[0].reshape(NS, e_rows // NS, CHUNK)
    dst_idx16 = edge_index[1].reshape(NS, e_rows // NS, CHUNK)
    ch = c // NC

    def _split(h):
        return jnp.stack([h[:, :ch], h[:, ch:]])

    nid_pad = NW * 128 * -(-n_dst // (NW * 128))
    nid3d = jnp.concatenate(
        [n_id_dst, jnp.zeros((nid_pad - n_dst,), jnp.int32)]
    ).reshape(NW, -1, 128)

    assert n_src == n_dst
    sc_pre = _make_sc_pre(vocab, c, nid_pad // 128, e_rows, n_dst)
    emb_rows, cnt_d, cnt_s = sc_pre(emb_shallow, nid3d, src_idx, dst_idx)
    emb_rows = emb_rows[:n_dst]
    cnt_d = cnt_d.reshape(NC, -1, LANE)[:, :n_dst]
    cnt_s = cnt_s.reshape(NC, -1, LANE)[:, :n_src]

    blk = 400
    grid = (n_src // blk,)
    row = functools.partial(_row_spec, blk)
    full = _full_spec
    cnt_spec = pl.BlockSpec((NC, blk, LANE), lambda i: (0, i, 0))
    part_spec = pl.BlockSpec((NC, blk, ch), lambda i: (0, i, 0))

    h_src, h_dst, inv_d, inv_s = pl.pallas_call(
        functools.partial(_enc_body, n_src, blk, bsz),
        grid=grid,
        in_specs=[row(d_in), row(d_in), row(1), full((1, bsz)), row(1),
                  row(c), cnt_spec, cnt_spec, full((d_in, c)), full((1, c)),
                  full((d_in, c)), full((1, c)), full((1, c)), full((1, c)),
                  full((1, c))],
        out_specs=[row(c), row(c), row(1), row(1)],
        out_shape=[jax.ShapeDtypeStruct((n_src, c), _F32),
                   jax.ShapeDtypeStruct((n_dst, c), _F32),
                   jax.ShapeDtypeStruct((n_dst, 1), _F32),
                   jax.ShapeDtypeStruct((n_src, 1), _F32)],
    )(x_src, x_dst, time_src[:, None], seed_time[None, :],
      batch_src[:, None], emb_rows, cnt_d, cnt_s,
      W_enc_src, b_enc_src[None, :], W_enc_dst, b_enc_dst[None, :],
      W_time, b_time[None, :], id_emb)

    agg2 = _make_sc_agg(n_dst, c, e_rows, 2)
    p_sd0, p_ds0 = agg2(_split(h_src), _split(h_dst), src_idx16, dst_idx16)
    p_sd0 = p_sd0.reshape(NC, -1, ch)[:, :n_dst]
    p_ds0 = p_ds0.reshape(NC, -1, ch)[:, :n_src]

    h_dst1, h_src1 = pl.pallas_call(
        _upd_body,
        grid=grid,
        in_specs=[row(c), row(c), part_spec, part_spec, row(1), row(1),
                  full((c, c)), full((c, c)), full((c, c)), full((c, c))],
        out_specs=[row(c), row(c)],
        out_shape=[jax.ShapeDtypeStruct((n_dst, c), _F32),
                   jax.ShapeDtypeStruct((n_src, c), _F32)],
    )(h_dst, h_src, p_sd0, p_ds0, inv_d, inv_s,
      W_root_d0, W_nbr_d0, W_root_s0, W_nbr_s0)

    agg1 = _make_sc_agg(n_dst, c, e_rows, 1)
    (p_sd1,) = agg1(_split(h_src1), src_idx16, dst_idx16)
    p_sd1 = p_sd1.reshape(NC, -1, ch)[:, :n_dst]

    out = pl.pallas_call(
        _fin_body,
        grid=grid,
        in_specs=[row(c), part_spec, row(1), full((c, c)), full((c, c)),
                  full((1, c)), full((1, c)), full((1, c)), full((1, 1))],
        out_specs=row(1),
        out_shape=jax.ShapeDtypeStruct((n_dst, 1), _F32),
    )(h_dst1, p_sd1, inv_d, W_root_d1, W_nbr_d1,
      ln_gamma[None, :], ln_beta[None, :], W_head.T, b_head[:, None])

    return out.reshape(-1)


# speedup vs baseline: 7.2746x; 1.2695x over previous
import functools

import jax
import jax.numpy as jnp
from jax import lax
from jax.experimental import pallas as pl
from jax.experimental.pallas import tpu as pltpu
from jax.experimental.pallas import tpu_sc as plsc

_SC_PARAMS = pltpu.CompilerParams(use_tc_tiling_on_sc=False)

NC = 2
NS = 16
NW = NC * NS
LANE = 16
CHUNK = 80
NBUF = 5

_F32 = jnp.float32
_HIGH = lax.Precision.HIGHEST


def _fill2d(ref, rows, cols, value):
    vec = jnp.full((LANE,), value, _F32)

    def row_body(i, _):
        def col_body(j, __):
            ref[i, pl.ds(j * LANE, LANE)] = vec
            return 0
        return lax.fori_loop(0, cols // LANE, col_body, 0)

    lax.fori_loop(0, rows, row_body, 0)


def _padded_rows(n):
    return 128 * -(-n // (NS * 128))


@functools.lru_cache(maxsize=None)
def _make_sc_pre(vocab, c, nid_rows, e_rows, n_nodes):
    emb_rows_w = nid_rows // NW
    er_w = e_rows // NW
    zt = _padded_rows(n_nodes)
    npad = NS * zt
    mesh = plsc.VectorSubcoreMesh(core_axis_name="c", subcore_axis_name="s")

    @functools.partial(
        pl.kernel,
        out_type=[
            jax.ShapeDtypeStruct((nid_rows * 128, c), _F32),
            jax.ShapeDtypeStruct((NC, NS, zt, LANE), _F32),
            jax.ShapeDtypeStruct((NC, NS, zt, LANE), _F32),
        ],
        mesh=mesh,
        compiler_params=_SC_PARAMS,
        scratch_types=[
            pltpu.VMEM((emb_rows_w, 128), jnp.int32),
            pltpu.VMEM((128, c), _F32),
            pltpu.VMEM((er_w, CHUNK), jnp.int32),
            pltpu.VMEM((er_w, CHUNK), jnp.int32),
            pltpu.VMEM((CHUNK, LANE), _F32),
            pltpu.VMEM((zt, LANE), _F32),
            pltpu.VMEM_SHARED((npad, LANE), _F32),
            pltpu.VMEM_SHARED((npad, LANE), _F32),
            pltpu.SemaphoreType.DMA,
        ],
    )
    def pre(emb_hbm, nid_hbm, src_hbm, dst_hbm,
            emb_out, cntd_out, cnts_out,
            nid_v, rows_v, src_v, dst_v, ones_v, zero_v, accd, accs, sem):
        cid = lax.axis_index("c")
        sid = lax.axis_index("s")
        w = cid * NS + sid

        pltpu.sync_copy(nid_hbm.at[w], nid_v)
        for r in range(emb_rows_w):
            pltpu.async_copy(emb_hbm.at[nid_v.at[r]], rows_v, sem).wait()
            pltpu.sync_copy(
                rows_v, emb_out.at[pl.ds((w * emb_rows_w + r) * 128, 128), :])

        _fill2d(ones_v, CHUNK, LANE, 1.0)
        _fill2d(zero_v, zt, LANE, 0.0)
        pltpu.sync_copy(src_hbm.at[w], src_v)
        pltpu.sync_copy(dst_hbm.at[w], dst_v)
        pltpu.sync_copy(zero_v, accd.at[pl.ds(sid * zt, zt), :])
        pltpu.sync_copy(zero_v, accs.at[pl.ds(sid * zt, zt), :])
        plsc.subcore_barrier()

        def cnt_body(i, _):
            pltpu.sync_copy(ones_v, accd.at[dst_v.at[i]], add=True)
            pltpu.sync_copy(ones_v, accs.at[src_v.at[i]], add=True)
            return 0

        lax.fori_loop(0, er_w, cnt_body, 0)
        plsc.subcore_barrier()
        pltpu.sync_copy(accd.at[pl.ds(sid * zt, zt), :], cntd_out.at[cid, sid])
        pltpu.sync_copy(accs.at[pl.ds(sid * zt, zt), :], cnts_out.at[cid, sid])

    return pre


@functools.lru_cache(maxsize=None)
def _make_sc_agg(n_nodes, c, e_rows, ndir):
    ch = c // NC
    er_w = e_rows // NS
    zt = _padded_rows(n_nodes)
    nz = zt // 128
    npad = NS * zt
    mesh = plsc.VectorSubcoreMesh(core_axis_name="c", subcore_axis_name="s")

    @functools.partial(
        pl.kernel,
        out_type=[jax.ShapeDtypeStruct((NC, NS * nz, 128, ch), _F32)
                  for _ in range(ndir)],
        mesh=mesh,
        compiler_params=_SC_PARAMS,
        scratch_types=[
            pltpu.VMEM((er_w, CHUNK), jnp.int32),
            pltpu.VMEM((er_w, CHUNK), jnp.int32),
        ] + [pltpu.VMEM((CHUNK, ch), _F32) for _ in range(NBUF)] + [
            pltpu.VMEM((128, ch), _F32),
            pltpu.VMEM_SHARED((npad, ch), _F32),
        ] + [pltpu.SemaphoreType.DMA for _ in range(NBUF)],
    )
    def agg(*refs):
        h_hbm = refs[0:ndir]
        src_hbm, dst_hbm = refs[ndir], refs[ndir + 1]
        outs = refs[ndir + 2: 2 * ndir + 2]
        rest = refs[2 * ndir + 2:]
        src_v, dst_v = rest[0], rest[1]
        rows = rest[2:2 + NBUF]
        zero_v, acc = rest[2 + NBUF], rest[3 + NBUF]
        sems = rest[4 + NBUF:]

        cid = lax.axis_index("c")
        sid = lax.axis_index("s")
        _fill2d(zero_v, 128, ch, 0.0)
        pltpu.sync_copy(src_hbm.at[sid], src_v)
        pltpu.sync_copy(dst_hbm.at[sid], dst_v)

        for d in range(ndir):
            gid_v = src_v if d == 0 else dst_v
            sid_v = dst_v if d == 0 else src_v
            for k in range(nz):
                pltpu.sync_copy(
                    zero_v, acc.at[pl.ds(sid * zt + k * 128, 128), :])
            plsc.subcore_barrier()

            h = h_hbm[d].at[cid]
            g, s = gid_v, sid_v

            def _wait(b, h=h, g=g):
                pltpu.make_async_copy(h.at[g.at[0]], rows[b], sems[b]).wait()

            for b in range(NBUF):
                pltpu.async_copy(h.at[g.at[b]], rows[b], sems[b])

            def ring_body(p, _, h=h, g=g, s=s):
                j = NBUF * p
                for b in range(NBUF):
                    _wait(b)
                    pltpu.sync_copy(rows[b], acc.at[s.at[j + b]], add=True)
                    pltpu.async_copy(h.at[g.at[j + b + NBUF]], rows[b],
                                     sems[b])
                return 0

            lax.fori_loop(0, er_w // NBUF - 1, ring_body, 0)
            for b in range(NBUF):
                _wait(b)
                pltpu.sync_copy(
                    rows[b], acc.at[s.at[er_w - NBUF + b]], add=True)
            plsc.subcore_barrier()
            for k in range(nz):
                sl = pl.ds(sid * zt + k * 128, 128)
                pltpu.sync_copy(acc.at[sl, :], outs[d].at[cid, sid * nz + k])
            if d + 1 < ndir:
                plsc.subcore_barrier()

    return agg


def _dot(a, b):
    return jnp.dot(a, b, preferred_element_type=_F32, precision=_HIGH)


def _enc_body(nrows, blk, bsz,
              xs_ref, xd_ref, ts_ref, st_ref, bsrc_ref, emb_ref,
              cntd_ref, cnts_ref, wes_ref, bes_ref, wed_ref, bed_ref,
              wt_ref, bt_ref, ide_ref,
              hs_out, hd_out, invd_out, invs_out):
    i = pl.program_id(0)
    hs = _dot(xs_ref[...], wes_ref[...]) + bes_ref[0][None, :]
    bs = bsrc_ref[...]
    cols = lax.broadcasted_iota(jnp.int32, (blk, bsz), 1)
    stg = jnp.sum(jnp.where(cols == bs, st_ref[...], 0.0), axis=1,
                  keepdims=True)
    rel = stg - ts_ref[...]
    hs = hs + rel * wt_ref[0][None, :] + bt_ref[0][None, :]
    row = i * blk + lax.broadcasted_iota(jnp.int32, (blk, 1), 0)
    hs = hs + jnp.where(row < bsz, 1.0, 0.0) * ide_ref[0][None, :]
    hs_out[...] = hs
    hd_out[...] = (_dot(xd_ref[...], wed_ref[...]) + bed_ref[0][None, :]
                   + emb_ref[...])
    cd = cntd_ref[0, :, 0] + cntd_ref[1, :, 0]
    cs = cnts_ref[0, :, 0] + cnts_ref[1, :, 0]
    invd_out[...] = (1.0 / jnp.maximum(cd, 1.0))[:, None]
    invs_out[...] = (1.0 / jnp.maximum(cs, 1.0))[:, None]


def _upd_body(hd_ref, hs_ref, psd_ref, pds_ref, invd_ref, invs_ref,
              wrd_ref, wnd_ref, wrs_ref, wns_ref, hd_out, hs_out):
    m_sd = jnp.concatenate([psd_ref[0], psd_ref[1]], axis=-1) * invd_ref[...]
    m_ds = jnp.concatenate([pds_ref[0], pds_ref[1]], axis=-1) * invs_ref[...]
    hd_out[...] = jnp.maximum(
        _dot(hd_ref[...], wrd_ref[...]) + _dot(m_sd, wnd_ref[...]), 0.0)
    hs_out[...] = jnp.maximum(
        _dot(hs_ref[...], wrs_ref[...]) + _dot(m_ds, wns_ref[...]), 0.0)


def _fin_body(hd_ref, psd_ref, invd_ref, wrd_ref, wnd_ref,
              gam_ref, bet_ref, wh_ref, bh_ref, out_ref):
    m_sd = jnp.concatenate([psd_ref[0], psd_ref[1]], axis=-1) * invd_ref[...]
    h2 = jnp.maximum(
        _dot(hd_ref[...], wrd_ref[...]) + _dot(m_sd, wnd_ref[...]), 0.0)
    mu = jnp.mean(h2, axis=-1, keepdims=True)
    var = jnp.mean((h2 - mu) ** 2, axis=-1, keepdims=True)
    hn = (h2 - mu) / jnp.sqrt(var + 1e-5) * gam_ref[0][None, :] \
        + bet_ref[0][None, :]
    out_ref[...] = jnp.sum(hn * wh_ref[0][None, :], axis=1,
                           keepdims=True) + bh_ref[0, 0]


def _row_spec(blk, width):
    return pl.BlockSpec((blk, width), lambda i: (i, 0))


def _full_spec(shape):
    return pl.BlockSpec(shape, lambda i: tuple(0 for _ in shape))


def kernel(x_src, x_dst, time_src, seed_time, W_enc_src, b_enc_src,
           W_enc_dst, b_enc_dst, W_time, b_time, emb_shallow, id_emb,
           W_root_s0, W_nbr_s0, W_root_d0, W_nbr_d0, W_root_s1, W_nbr_s1,
           W_root_d1, W_nbr_d1, ln_gamma, ln_beta, W_head, b_head,
           batch_src, n_id_dst, edge_index):
    n_src, d_in = x_src.shape
    n_dst = x_dst.shape[0]
    c = W_enc_src.shape[1]
    e = edge_index.shape[1]
    bsz = seed_time.shape[0]
    vocab = emb_shallow.shape[0]
    assert e % (NW * CHUNK) == 0 and n_dst % (NS * 5) == 0 and n_src % (NS * 5) == 0

    e_rows = e // CHUNK
    src_idx = edge_index[0].reshape(NW, e_rows // NW, CHUNK)
    dst_idx = edge_index[1].reshape(NW, e_rows // NW, CHUNK)
    src_idx16 = edge_index[0].reshape(NS, e_rows // NS, CHUNK)
    dst_idx16 = edge_index[1].reshape(NS, e_rows // NS, CHUNK)
    ch = c // NC

    def _split(h):
        return jnp.stack([h[:, :ch], h[:, ch:]])

    nid_pad = NW * 128 * -(-n_dst // (NW * 128))
    nid3d = jnp.concatenate(
        [n_id_dst, jnp.zeros((nid_pad - n_dst,), jnp.int32)]
    ).reshape(NW, -1, 128)

    assert n_src == n_dst
    sc_pre = _make_sc_pre(vocab, c, nid_pad // 128, e_rows, n_dst)
    emb_rows, cnt_d, cnt_s = sc_pre(emb_shallow, nid3d, src_idx, dst_idx)
    emb_rows = emb_rows[:n_dst]
    cnt_d = cnt_d.reshape(NC, -1, LANE)[:, :n_dst]
    cnt_s = cnt_s.reshape(NC, -1, LANE)[:, :n_src]

    blk = 400
    grid = (n_src // blk,)
    row = functools.partial(_row_spec, blk)
    full = _full_spec
    cnt_spec = pl.BlockSpec((NC, blk, LANE), lambda i: (0, i, 0))
    part_spec = pl.BlockSpec((NC, blk, ch), lambda i: (0, i, 0))

    h_src, h_dst, inv_d, inv_s = pl.pallas_call(
        functools.partial(_enc_body, n_src, blk, bsz),
        grid=grid,
        in_specs=[row(d_in), row(d_in), row(1), full((1, bsz)), row(1),
                  row(c), cnt_spec, cnt_spec, full((d_in, c)), full((1, c)),
                  full((d_in, c)), full((1, c)), full((1, c)), full((1, c)),
                  full((1, c))],
        out_specs=[row(c), row(c), row(1), row(1)],
        out_shape=[jax.ShapeDtypeStruct((n_src, c), _F32),
                   jax.ShapeDtypeStruct((n_dst, c), _F32),
                   jax.ShapeDtypeStruct((n_dst, 1), _F32),
                   jax.ShapeDtypeStruct((n_src, 1), _F32)],
    )(x_src, x_dst, time_src[:, None], seed_time[None, :],
      batch_src[:, None], emb_rows, cnt_d, cnt_s,
      W_enc_src, b_enc_src[None, :], W_enc_dst, b_enc_dst[None, :],
      W_time, b_time[None, :], id_emb)

    agg2 = _make_sc_agg(n_dst, c, e_rows, 2)
    p_sd0, p_ds0 = agg2(_split(h_src), _split(h_dst), src_idx16, dst_idx16)
    p_sd0 = p_sd0.reshape(NC, -1, ch)[:, :n_dst]
    p_ds0 = p_ds0.reshape(NC, -1, ch)[:, :n_src]

    h_dst1, h_src1 = pl.pallas_call(
        _upd_body,
        grid=grid,
        in_specs=[row(c), row(c), part_spec, part_spec, row(1), row(1),
                  full((c, c)), full((c, c)), full((c, c)), full((c, c))],
        out_specs=[row(c), row(c)],
        out_shape=[jax.ShapeDtypeStruct((n_dst, c), _F32),
                   jax.ShapeDtypeStruct((n_src, c), _F32)],
    )(h_dst, h_src, p_sd0, p_ds0, inv_d, inv_s,
      W_root_d0, W_nbr_d0, W_root_s0, W_nbr_s0)

    agg1 = _make_sc_agg(n_dst, c, e_rows, 1)
    (p_sd1,) = agg1(_split(h_src1), src_idx16, dst_idx16)
    p_sd1 = p_sd1.reshape(NC, -1, ch)[:, :n_dst]

    out = pl.pallas_call(
        _fin_body,
        grid=grid,
        in_specs=[row(c), part_spec, row(1), full((c, c)), full((c, c)),
                  full((1, c)), full((1, c)), full((1, c)), full((1, 1))],
        out_specs=row(1),
        out_shape=jax.ShapeDtypeStruct((n_dst, 1), _F32),
    )(h_dst1, p_sd1, inv_d, W_root_d1, W_nbr_d1,
      ln_gamma[None, :], ln_beta[None, :], W_head.T, b_head[:, None])

    return out.reshape(-1)


# speedup vs baseline: 8.1762x; 1.1239x over previous
import functools

import jax
import jax.numpy as jnp
from jax import lax
from jax.experimental import pallas as pl
from jax.experimental.pallas import tpu as pltpu
from jax.experimental.pallas import tpu_sc as plsc

_SC_PARAMS = pltpu.CompilerParams(use_tc_tiling_on_sc=False)

NC = 2
NS = 16
NW = NC * NS
LANE = 16
CHUNK = 80
NBUF = 5

_F32 = jnp.float32
_HIGH = lax.Precision.HIGHEST


def _fill2d(ref, rows, cols, value):
    vec = jnp.full((LANE,), value, _F32)

    def row_body(i, _):
        def col_body(j, __):
            ref[i, pl.ds(j * LANE, LANE)] = vec
            return 0
        return lax.fori_loop(0, cols // LANE, col_body, 0)

    lax.fori_loop(0, rows, row_body, 0)


def _padded_rows(n):
    return 128 * -(-n // (NS * 128))


@functools.lru_cache(maxsize=None)
def _make_sc_pre(vocab, c, nid_rows, e_rows, n_nodes):
    emb_rows_w = nid_rows // NW
    er_w = e_rows // NW
    zt = _padded_rows(n_nodes)
    npad = NS * zt
    mesh = plsc.VectorSubcoreMesh(core_axis_name="c", subcore_axis_name="s")

    @functools.partial(
        pl.kernel,
        out_type=[
            jax.ShapeDtypeStruct((nid_rows * 128, c), _F32),
            jax.ShapeDtypeStruct((NC, NS, zt, LANE), _F32),
            jax.ShapeDtypeStruct((NC, NS, zt, LANE), _F32),
        ],
        mesh=mesh,
        compiler_params=_SC_PARAMS,
        scratch_types=[
            pltpu.VMEM((emb_rows_w, 128), jnp.int32),
            pltpu.VMEM((128, c), _F32),
            pltpu.VMEM((er_w, CHUNK), jnp.int32),
            pltpu.VMEM((er_w, CHUNK), jnp.int32),
            pltpu.VMEM((CHUNK, LANE), _F32),
            pltpu.VMEM((zt, LANE), _F32),
            pltpu.VMEM_SHARED((npad, LANE), _F32),
            pltpu.VMEM_SHARED((npad, LANE), _F32),
            pltpu.SemaphoreType.DMA,
        ],
    )
    def pre(emb_hbm, nid_hbm, src_hbm, dst_hbm,
            emb_out, cntd_out, cnts_out,
            nid_v, rows_v, src_v, dst_v, ones_v, zero_v, accd, accs, sem):
        cid = lax.axis_index("c")
        sid = lax.axis_index("s")
        w = cid * NS + sid

        pltpu.sync_copy(nid_hbm.at[w], nid_v)
        for r in range(emb_rows_w):
            pltpu.async_copy(emb_hbm.at[nid_v.at[r]], rows_v, sem).wait()
            pltpu.sync_copy(
                rows_v, emb_out.at[pl.ds((w * emb_rows_w + r) * 128, 128), :])

        _fill2d(ones_v, CHUNK, LANE, 1.0)
        _fill2d(zero_v, zt, LANE, 0.0)
        pltpu.sync_copy(src_hbm.at[w], src_v)
        pltpu.sync_copy(dst_hbm.at[w], dst_v)
        pltpu.sync_copy(zero_v, accd.at[pl.ds(sid * zt, zt), :])
        pltpu.sync_copy(zero_v, accs.at[pl.ds(sid * zt, zt), :])
        plsc.subcore_barrier()

        def cnt_body(i, _):
            pltpu.sync_copy(ones_v, accd.at[dst_v.at[i]], add=True)
            pltpu.sync_copy(ones_v, accs.at[src_v.at[i]], add=True)
            return 0

        lax.fori_loop(0, er_w, cnt_body, 0)
        plsc.subcore_barrier()
        pltpu.sync_copy(accd.at[pl.ds(sid * zt, zt), :], cntd_out.at[cid, sid])
        pltpu.sync_copy(accs.at[pl.ds(sid * zt, zt), :], cnts_out.at[cid, sid])

    return pre


@functools.lru_cache(maxsize=None)
def _make_sc_agg(n_nodes, c, e_rows, ndir):
    ch = c // NC
    er_w = e_rows // NS
    zt = _padded_rows(n_nodes)
    nz = zt // 128
    npad = NS * zt
    mesh = plsc.VectorSubcoreMesh(core_axis_name="c", subcore_axis_name="s")

    @functools.partial(
        pl.kernel,
        out_type=[jax.ShapeDtypeStruct((NC, NS * nz, 128, ch), _F32)
                  for _ in range(ndir)],
        mesh=mesh,
        compiler_params=_SC_PARAMS,
        scratch_types=[
            pltpu.VMEM((er_w, CHUNK), jnp.int32),
            pltpu.VMEM((er_w, CHUNK), jnp.int32),
        ] + [pltpu.VMEM((CHUNK, ch), _F32) for _ in range(NBUF)] + [
            pltpu.VMEM((128, ch), _F32),
            pltpu.VMEM_SHARED((npad, ch), _F32),
        ] + [pltpu.SemaphoreType.DMA for _ in range(NBUF)],
    )
    def agg(*refs):
        h_hbm = refs[0:ndir]
        src_hbm, dst_hbm = refs[ndir], refs[ndir + 1]
        outs = refs[ndir + 2: 2 * ndir + 2]
        rest = refs[2 * ndir + 2:]
        src_v, dst_v = rest[0], rest[1]
        rows = rest[2:2 + NBUF]
        zero_v, acc = rest[2 + NBUF], rest[3 + NBUF]
        sems = rest[4 + NBUF:]

        cid = lax.axis_index("c")
        sid = lax.axis_index("s")
        _fill2d(zero_v, 128, ch, 0.0)
        pltpu.sync_copy(src_hbm.at[sid], src_v)
        pltpu.sync_copy(dst_hbm.at[sid], dst_v)

        for d in range(ndir):
            gid_v = src_v if d == 0 else dst_v
            sid_v = dst_v if d == 0 else src_v
            for k in range(nz):
                pltpu.sync_copy(
                    zero_v, acc.at[pl.ds(sid * zt + k * 128, 128), :])
            plsc.subcore_barrier()

            h = h_hbm[d].at[cid]
            g, s = gid_v, sid_v

            def _wait(b, h=h, g=g):
                pltpu.make_async_copy(h.at[g.at[0]], rows[b], sems[b]).wait()

            for b in range(NBUF):
                pltpu.async_copy(h.at[g.at[b]], rows[b], sems[b])

            def ring_body(p, _, h=h, g=g, s=s):
                j = NBUF * p
                for b in range(NBUF):
                    _wait(b)
                    pltpu.sync_copy(rows[b], acc.at[s.at[j + b]], add=True)
                    pltpu.async_copy(h.at[g.at[j + b + NBUF]], rows[b],
                                     sems[b])
                return 0

            lax.fori_loop(0, er_w // NBUF - 1, ring_body, 0)
            for b in range(NBUF):
                _wait(b)
                pltpu.sync_copy(
                    rows[b], acc.at[s.at[er_w - NBUF + b]], add=True)
            plsc.subcore_barrier()
            for k in range(nz):
                sl = pl.ds(sid * zt + k * 128, 128)
                pltpu.sync_copy(acc.at[sl, :], outs[d].at[cid, sid * nz + k])
            if d + 1 < ndir:
                plsc.subcore_barrier()

    return agg


def _dot(a, b):
    return jnp.dot(a, b, preferred_element_type=_F32, precision=_HIGH)


def _enc_body(nrows, blk, bsz, ch,
              xs_ref, xd_ref, ts_ref, st_ref, bsrc_ref, emb_ref,
              wes_ref, bes_ref, wed_ref, bed_ref,
              wt_ref, bt_ref, ide_ref,
              hs_out, hd_out, hss_out, hds_out):
    i = pl.program_id(0)
    hs = _dot(xs_ref[...], wes_ref[...]) + bes_ref[0][None, :]
    bs = bsrc_ref[...]
    cols = lax.broadcasted_iota(jnp.int32, (blk, bsz), 1)
    stg = jnp.sum(jnp.where(cols == bs, st_ref[...], 0.0), axis=1,
                  keepdims=True)
    rel = stg - ts_ref[...]
    hs = hs + rel * wt_ref[0][None, :] + bt_ref[0][None, :]
    row = i * blk + lax.broadcasted_iota(jnp.int32, (blk, 1), 0)
    hs = hs + jnp.where(row < bsz, 1.0, 0.0) * ide_ref[0][None, :]
    hs_out[...] = hs
    hd = (_dot(xd_ref[...], wed_ref[...]) + bed_ref[0][None, :]
          + emb_ref[...])
    hd_out[...] = hd
    hss_out[0] = hs[:, :ch]
    hss_out[1] = hs[:, ch:]
    hds_out[0] = hd[:, :ch]
    hds_out[1] = hd[:, ch:]


def _inv(cnt_ref):
    return (1.0 / jnp.maximum(cnt_ref[0, :, 0] + cnt_ref[1, :, 0],
                              1.0))[:, None]


def _upd_body(ch, hd_ref, hs_ref, psd_ref, pds_ref, cntd_ref, cnts_ref,
              wrd_ref, wnd_ref, wrs_ref, wns_ref, hd_out, hss_out):
    m_sd = jnp.concatenate([psd_ref[0], psd_ref[1]], axis=-1) * _inv(cntd_ref)
    m_ds = jnp.concatenate([pds_ref[0], pds_ref[1]], axis=-1) * _inv(cnts_ref)
    hd_out[...] = jnp.maximum(
        _dot(hd_ref[...], wrd_ref[...]) + _dot(m_sd, wnd_ref[...]), 0.0)
    hs1 = jnp.maximum(
        _dot(hs_ref[...], wrs_ref[...]) + _dot(m_ds, wns_ref[...]), 0.0)
    hss_out[0] = hs1[:, :ch]
    hss_out[1] = hs1[:, ch:]


def _fin_body(hd_ref, psd_ref, cntd_ref, wrd_ref, wnd_ref,
              gam_ref, bet_ref, wh_ref, bh_ref, out_ref):
    m_sd = jnp.concatenate([psd_ref[0], psd_ref[1]], axis=-1) * _inv(cntd_ref)
    h2 = jnp.maximum(
        _dot(hd_ref[...], wrd_ref[...]) + _dot(m_sd, wnd_ref[...]), 0.0)
    mu = jnp.mean(h2, axis=-1, keepdims=True)
    var = jnp.mean((h2 - mu) ** 2, axis=-1, keepdims=True)
    hn = (h2 - mu) / jnp.sqrt(var + 1e-5) * gam_ref[0][None, :] \
        + bet_ref[0][None, :]
    out_ref[...] = jnp.sum(hn * wh_ref[0][None, :], axis=1,
                           keepdims=True) + bh_ref[0, 0]


def _row_spec(blk, width):
    return pl.BlockSpec((blk, width), lambda i: (i, 0))


def _full_spec(shape):
    return pl.BlockSpec(shape, lambda i: tuple(0 for _ in shape))


def kernel(x_src, x_dst, time_src, seed_time, W_enc_src, b_enc_src,
           W_enc_dst, b_enc_dst, W_time, b_time, emb_shallow, id_emb,
           W_root_s0, W_nbr_s0, W_root_d0, W_nbr_d0, W_root_s1, W_nbr_s1,
           W_root_d1, W_nbr_d1, ln_gamma, ln_beta, W_head, b_head,
           batch_src, n_id_dst, edge_index):
    n_src, d_in = x_src.shape
    n_dst = x_dst.shape[0]
    c = W_enc_src.shape[1]
    e = edge_index.shape[1]
    bsz = seed_time.shape[0]
    vocab = emb_shallow.shape[0]
    assert e % (NW * CHUNK) == 0 and n_dst % (NS * 5) == 0 and n_src % (NS * 5) == 0

    e_rows = e // CHUNK
    src_idx = edge_index[0].reshape(NW, e_rows // NW, CHUNK)
    dst_idx = edge_index[1].reshape(NW, e_rows // NW, CHUNK)
    src_idx16 = edge_index[0].reshape(NS, e_rows // NS, CHUNK)
    dst_idx16 = edge_index[1].reshape(NS, e_rows // NS, CHUNK)
    ch = c // NC

    nid_pad = NW * 128 * -(-n_dst // (NW * 128))
    nid3d = jnp.concatenate(
        [n_id_dst, jnp.zeros((nid_pad - n_dst,), jnp.int32)]
    ).reshape(NW, -1, 128)

    assert n_src == n_dst
    sc_pre = _make_sc_pre(vocab, c, nid_pad // 128, e_rows, n_dst)
    emb_rows, cnt_d, cnt_s = sc_pre(emb_shallow, nid3d, src_idx, dst_idx)
    cnt_d = cnt_d.reshape(NC, -1, LANE)
    cnt_s = cnt_s.reshape(NC, -1, LANE)

    blk = 400
    grid = (n_src // blk,)
    row = functools.partial(_row_spec, blk)
    full = _full_spec
    cnt_spec = pl.BlockSpec((NC, blk, LANE), lambda i: (0, i, 0))
    part_spec = pl.BlockSpec((NC, blk, ch), lambda i: (0, i, 0))
    split_spec = pl.BlockSpec((NC, blk, ch), lambda i: (0, i, 0))

    h_src, h_dst, hs_sp, hd_sp = pl.pallas_call(
        functools.partial(_enc_body, n_src, blk, bsz, ch),
        grid=grid,
        in_specs=[row(d_in), row(d_in), row(1), full((1, bsz)), row(1),
                  row(c), full((d_in, c)), full((1, c)),
                  full((d_in, c)), full((1, c)), full((1, c)), full((1, c)),
                  full((1, c))],
        out_specs=[row(c), row(c), split_spec, split_spec],
        out_shape=[jax.ShapeDtypeStruct((n_src, c), _F32),
                   jax.ShapeDtypeStruct((n_dst, c), _F32),
                   jax.ShapeDtypeStruct((NC, n_src, ch), _F32),
                   jax.ShapeDtypeStruct((NC, n_dst, ch), _F32)],
    )(x_src, x_dst, time_src[:, None], seed_time[None, :],
      batch_src[:, None], emb_rows,
      W_enc_src, b_enc_src[None, :], W_enc_dst, b_enc_dst[None, :],
      W_time, b_time[None, :], id_emb)

    agg2 = _make_sc_agg(n_dst, c, e_rows, 2)
    p_sd0, p_ds0 = agg2(hs_sp, hd_sp, src_idx16, dst_idx16)
    p_sd0 = p_sd0.reshape(NC, -1, ch)
    p_ds0 = p_ds0.reshape(NC, -1, ch)

    h_dst1, hs1_sp = pl.pallas_call(
        functools.partial(_upd_body, ch),
        grid=grid,
        in_specs=[row(c), row(c), part_spec, part_spec, cnt_spec, cnt_spec,
                  full((c, c)), full((c, c)), full((c, c)), full((c, c))],
        out_specs=[row(c), split_spec],
        out_shape=[jax.ShapeDtypeStruct((n_dst, c), _F32),
                   jax.ShapeDtypeStruct((NC, n_src, ch), _F32)],
    )(h_dst, h_src, p_sd0, p_ds0, cnt_d, cnt_s,
      W_root_d0, W_nbr_d0, W_root_s0, W_nbr_s0)

    agg1 = _make_sc_agg(n_dst, c, e_rows, 1)
    (p_sd1,) = agg1(hs1_sp, src_idx16, dst_idx16)
    p_sd1 = p_sd1.reshape(NC, -1, ch)

    out = pl.pallas_call(
        _fin_body,
        grid=grid,
        in_specs=[row(c), part_spec, cnt_spec, full((c, c)), full((c, c)),
                  full((1, c)), full((1, c)), full((1, c)), full((1, 1))],
        out_specs=row(1),
        out_shape=jax.ShapeDtypeStruct((n_dst, 1), _F32),
    )(h_dst1, p_sd1, cnt_d, W_root_d1, W_nbr_d1,
      ln_gamma[None, :], ln_beta[None, :], W_head.T, b_head[:, None])

    return out.reshape(-1)


# speedup vs baseline: 8.3924x; 1.0264x over previous
import functools

import jax
import jax.numpy as jnp
from jax import lax
from jax.experimental import pallas as pl
from jax.experimental.pallas import tpu as pltpu
from jax.experimental.pallas import tpu_sc as plsc

_SC_PARAMS = pltpu.CompilerParams(use_tc_tiling_on_sc=False)

NC = 2
NS = 16
NW = NC * NS
LANE = 16
CHUNK = 125
NBUF = 5

_F32 = jnp.float32
_HIGH = lax.Precision.HIGHEST


def _fill2d(ref, rows, cols, value):
    vec = jnp.full((LANE,), value, _F32)

    def row_body(i, _):
        def col_body(j, __):
            ref[i, pl.ds(j * LANE, LANE)] = vec
            return 0
        return lax.fori_loop(0, cols // LANE, col_body, 0)

    lax.fori_loop(0, rows, row_body, 0)


def _padded_rows(n):
    return 128 * -(-n // (NS * 128))


@functools.lru_cache(maxsize=None)
def _make_sc_pre(vocab, c, nid_rows, e_rows, n_nodes):
    emb_rows_w = nid_rows // NW
    er_w = e_rows // NW
    zt = _padded_rows(n_nodes)
    npad = NS * zt
    mesh = plsc.VectorSubcoreMesh(core_axis_name="c", subcore_axis_name="s")

    @functools.partial(
        pl.kernel,
        out_type=[
            jax.ShapeDtypeStruct((nid_rows * 128, c), _F32),
            jax.ShapeDtypeStruct((NC, NS, zt, LANE), _F32),
            jax.ShapeDtypeStruct((NC, NS, zt, LANE), _F32),
        ],
        mesh=mesh,
        compiler_params=_SC_PARAMS,
        scratch_types=[
            pltpu.VMEM((emb_rows_w, 128), jnp.int32),
            pltpu.VMEM((128, c), _F32),
            pltpu.VMEM((128, c), _F32),
            pltpu.VMEM((er_w, CHUNK), jnp.int32),
            pltpu.VMEM((er_w, CHUNK), jnp.int32),
            pltpu.VMEM((CHUNK, LANE), _F32),
            pltpu.VMEM((zt, LANE), _F32),
            pltpu.VMEM_SHARED((npad, LANE), _F32),
            pltpu.VMEM_SHARED((npad, LANE), _F32),
            pltpu.SemaphoreType.DMA,
            pltpu.SemaphoreType.DMA,
            pltpu.SemaphoreType.DMA,
            pltpu.SemaphoreType.DMA,
        ],
    )
    def pre(emb_hbm, nid_hbm, src_hbm, dst_hbm,
            emb_out, cntd_out, cnts_out,
            nid_v, rows_a, rows_b, src_v, dst_v, ones_v, zero_v, accd, accs,
            sem_a, sem_b, sem_d, sem_s):
        cid = lax.axis_index("c")
        sid = lax.axis_index("s")
        w = cid * NS + sid

        _fill2d(ones_v, CHUNK, LANE, 1.0)
        _fill2d(zero_v, zt, LANE, 0.0)
        pltpu.sync_copy(src_hbm.at[w], src_v)
        pltpu.sync_copy(dst_hbm.at[w], dst_v)
        pltpu.sync_copy(zero_v, accd.at[pl.ds(sid * zt, zt), :])
        pltpu.sync_copy(zero_v, accs.at[pl.ds(sid * zt, zt), :])
        plsc.subcore_barrier()

        def cnt_body(i, _):
            pltpu.async_copy(ones_v, accd.at[dst_v.at[i]], sem_d, add=True)
            pltpu.async_copy(ones_v, accs.at[src_v.at[i]], sem_s, add=True)
            return 0

        lax.fori_loop(0, er_w, cnt_body, 0)

        pltpu.sync_copy(nid_hbm.at[w], nid_v)
        rings = [(rows_a, sem_a), (rows_b, sem_b)]
        for r in range(min(2, emb_rows_w)):
            buf, sem = rings[r % 2]
            pltpu.async_copy(emb_hbm.at[nid_v.at[r]], buf, sem)
        for r in range(emb_rows_w):
            buf, sem = rings[r % 2]
            pltpu.make_async_copy(emb_hbm.at[nid_v.at[0]], buf, sem).wait()
            pltpu.sync_copy(
                buf, emb_out.at[pl.ds((w * emb_rows_w + r) * 128, 128), :])
            if r + 2 < emb_rows_w:
                pltpu.async_copy(emb_hbm.at[nid_v.at[r + 2]], buf, sem)

        def cnt_drain(i, _):
            pltpu.make_async_copy(ones_v, accd.at[dst_v.at[0]], sem_d).wait()
            pltpu.make_async_copy(ones_v, accs.at[src_v.at[0]], sem_s).wait()
            return 0

        lax.fori_loop(0, er_w, cnt_drain, 0)
        plsc.subcore_barrier()
        pltpu.sync_copy(accd.at[pl.ds(sid * zt, zt), :], cntd_out.at[cid, sid])
        pltpu.sync_copy(accs.at[pl.ds(sid * zt, zt), :], cnts_out.at[cid, sid])

    return pre


@functools.lru_cache(maxsize=None)
def _make_sc_agg(n_nodes, c, e_rows, ndir):
    ch = c // NC
    er_w = e_rows // NS
    zt = _padded_rows(n_nodes)
    nz = zt // 128
    npad = NS * zt
    mesh = plsc.VectorSubcoreMesh(core_axis_name="c", subcore_axis_name="s")

    @functools.partial(
        pl.kernel,
        out_type=[jax.ShapeDtypeStruct((NC, NS * nz, 128, ch), _F32)
                  for _ in range(ndir)],
        mesh=mesh,
        compiler_params=_SC_PARAMS,
        scratch_types=[
            pltpu.VMEM((er_w, CHUNK), jnp.int32),
            pltpu.VMEM((er_w, CHUNK), jnp.int32),
        ] + [pltpu.VMEM((CHUNK, ch), _F32) for _ in range(NBUF)] + [
            pltpu.VMEM((128, ch), _F32),
            pltpu.VMEM_SHARED((npad, ch), _F32),
        ] + [pltpu.SemaphoreType.DMA for _ in range(NBUF)],
    )
    def agg(*refs):
        h_hbm = refs[0:ndir]
        src_hbm, dst_hbm = refs[ndir], refs[ndir + 1]
        outs = refs[ndir + 2: 2 * ndir + 2]
        rest = refs[2 * ndir + 2:]
        src_v, dst_v = rest[0], rest[1]
        rows = rest[2:2 + NBUF]
        zero_v, acc = rest[2 + NBUF], rest[3 + NBUF]
        sems = rest[4 + NBUF:]

        cid = lax.axis_index("c")
        sid = lax.axis_index("s")
        _fill2d(zero_v, 128, ch, 0.0)
        pltpu.sync_copy(src_hbm.at[sid], src_v)
        pltpu.sync_copy(dst_hbm.at[sid], dst_v)

        for d in range(ndir):
            gid_v = src_v if d == 0 else dst_v
            sid_v = dst_v if d == 0 else src_v
            for k in range(nz):
                pltpu.sync_copy(
                    zero_v, acc.at[pl.ds(sid * zt + k * 128, 128), :])
            plsc.subcore_barrier()

            h = h_hbm[d].at[cid]
            g, s = gid_v, sid_v

            def _wait(b, h=h, g=g):
                pltpu.make_async_copy(h.at[g.at[0]], rows[b], sems[b]).wait()

            for b in range(NBUF):
                pltpu.async_copy(h.at[g.at[b]], rows[b], sems[b])

            def ring_body(p, _, h=h, g=g, s=s):
                j = NBUF * p
                for b in range(NBUF):
                    _wait(b)
                    pltpu.sync_copy(rows[b], acc.at[s.at[j + b]], add=True)
                    pltpu.async_copy(h.at[g.at[j + b + NBUF]], rows[b],
                                     sems[b])
                return 0

            lax.fori_loop(0, er_w // NBUF - 1, ring_body, 0)
            for b in range(NBUF):
                _wait(b)
                pltpu.sync_copy(
                    rows[b], acc.at[s.at[er_w - NBUF + b]], add=True)
            plsc.subcore_barrier()
            for k in range(nz):
                sl = pl.ds(sid * zt + k * 128, 128)
                pltpu.sync_copy(acc.at[sl, :], outs[d].at[cid, sid * nz + k])
            if d + 1 < ndir:
                plsc.subcore_barrier()

    return agg


def _dot(a, b):
    return jnp.dot(a, b, preferred_element_type=_F32, precision=_HIGH)


def _enc_body(nrows, blk, bsz, ch,
              xs_ref, xd_ref, ts_ref, st_ref, bsrc_ref, emb_ref,
              wes_ref, bes_ref, wed_ref, bed_ref,
              wt_ref, bt_ref, ide_ref,
              hs_out, hd_out, hss_out, hds_out):
    i = pl.program_id(0)
    hs = _dot(xs_ref[...], wes_ref[...]) + bes_ref[0][None, :]
    bs = bsrc_ref[...]
    cols = lax.broadcasted_iota(jnp.int32, (blk, bsz), 1)
    stg = jnp.sum(jnp.where(cols == bs, st_ref[...], 0.0), axis=1,
                  keepdims=True)
    rel = stg - ts_ref[...]
    hs = hs + rel * wt_ref[0][None, :] + bt_ref[0][None, :]
    row = i * blk + lax.broadcasted_iota(jnp.int32, (blk, 1), 0)
    hs = hs + jnp.where(row < bsz, 1.0, 0.0) * ide_ref[0][None, :]
    hs_out[...] = hs
    hd = (_dot(xd_ref[...], wed_ref[...]) + bed_ref[0][None, :]
          + emb_ref[...])
    hd_out[...] = hd
    hss_out[0] = hs[:, :ch]
    hss_out[1] = hs[:, ch:]
    hds_out[0] = hd[:, :ch]
    hds_out[1] = hd[:, ch:]


def _inv(cnt_ref):
    return (1.0 / jnp.maximum(cnt_ref[0, :, 0] + cnt_ref[1, :, 0],
                              1.0))[:, None]


def _upd_body(ch, hd_ref, hs_ref, psd_ref, pds_ref, cntd_ref, cnts_ref,
              wrd_ref, wnd_ref, wrs_ref, wns_ref, hd_out, hss_out):
    m_sd = jnp.concatenate([psd_ref[0], psd_ref[1]], axis=-1) * _inv(cntd_ref)
    m_ds = jnp.concatenate([pds_ref[0], pds_ref[1]], axis=-1) * _inv(cnts_ref)
    hd_out[...] = jnp.maximum(
        _dot(hd_ref[...], wrd_ref[...]) + _dot(m_sd, wnd_ref[...]), 0.0)
    hs1 = jnp.maximum(
        _dot(hs_ref[...], wrs_ref[...]) + _dot(m_ds, wns_ref[...]), 0.0)
    hss_out[0] = hs1[:, :ch]
    hss_out[1] = hs1[:, ch:]


def _fin_body(hd_ref, psd_ref, cntd_ref, wrd_ref, wnd_ref,
              gam_ref, bet_ref, wh_ref, bh_ref, out_ref):
    m_sd = jnp.concatenate([psd_ref[0], psd_ref[1]], axis=-1) * _inv(cntd_ref)
    h2 = jnp.maximum(
        _dot(hd_ref[...], wrd_ref[...]) + _dot(m_sd, wnd_ref[...]), 0.0)
    mu = jnp.mean(h2, axis=-1, keepdims=True)
    var = jnp.mean((h2 - mu) ** 2, axis=-1, keepdims=True)
    hn = (h2 - mu) / jnp.sqrt(var + 1e-5) * gam_ref[0][None, :] \
        + bet_ref[0][None, :]
    out_ref[...] = jnp.sum(hn * wh_ref[0][None, :], axis=1,
                           keepdims=True) + bh_ref[0, 0]


def _row_spec(blk, width):
    return pl.BlockSpec((blk, width), lambda i: (i, 0))


def _full_spec(shape):
    return pl.BlockSpec(shape, lambda i: tuple(0 for _ in shape))


def kernel(x_src, x_dst, time_src, seed_time, W_enc_src, b_enc_src,
           W_enc_dst, b_enc_dst, W_time, b_time, emb_shallow, id_emb,
           W_root_s0, W_nbr_s0, W_root_d0, W_nbr_d0, W_root_s1, W_nbr_s1,
           W_root_d1, W_nbr_d1, ln_gamma, ln_beta, W_head, b_head,
           batch_src, n_id_dst, edge_index):
    n_src, d_in = x_src.shape
    n_dst = x_dst.shape[0]
    c = W_enc_src.shape[1]
    e = edge_index.shape[1]
    bsz = seed_time.shape[0]
    vocab = emb_shallow.shape[0]
    assert e % (NW * CHUNK) == 0 and n_dst % (NS * 5) == 0 and n_src % (NS * 5) == 0

    e_rows = e // CHUNK
    src_idx = edge_index[0].reshape(NW, e_rows // NW, CHUNK)
    dst_idx = edge_index[1].reshape(NW, e_rows // NW, CHUNK)
    src_idx16 = edge_index[0].reshape(NS, e_rows // NS, CHUNK)
    dst_idx16 = edge_index[1].reshape(NS, e_rows // NS, CHUNK)
    ch = c // NC

    nid_pad = NW * 128 * -(-n_dst // (NW * 128))
    nid3d = jnp.concatenate(
        [n_id_dst, jnp.zeros((nid_pad - n_dst,), jnp.int32)]
    ).reshape(NW, -1, 128)

    assert n_src == n_dst
    sc_pre = _make_sc_pre(vocab, c, nid_pad // 128, e_rows, n_dst)
    emb_rows, cnt_d, cnt_s = sc_pre(emb_shallow, nid3d, src_idx, dst_idx)
    cnt_d = cnt_d.reshape(NC, -1, LANE)
    cnt_s = cnt_s.reshape(NC, -1, LANE)

    blk = 400
    grid = (n_src // blk,)
    row = functools.partial(_row_spec, blk)
    full = _full_spec
    cnt_spec = pl.BlockSpec((NC, blk, LANE), lambda i: (0, i, 0))
    part_spec = pl.BlockSpec((NC, blk, ch), lambda i: (0, i, 0))
    split_spec = pl.BlockSpec((NC, blk, ch), lambda i: (0, i, 0))

    h_src, h_dst, hs_sp, hd_sp = pl.pallas_call(
        functools.partial(_enc_body, n_src, blk, bsz, ch),
        grid=grid,
        in_specs=[row(d_in), row(d_in), row(1), full((1, bsz)), row(1),
                  row(c), full((d_in, c)), full((1, c)),
                  full((d_in, c)), full((1, c)), full((1, c)), full((1, c)),
                  full((1, c))],
        out_specs=[row(c), row(c), split_spec, split_spec],
        out_shape=[jax.ShapeDtypeStruct((n_src, c), _F32),
                   jax.ShapeDtypeStruct((n_dst, c), _F32),
                   jax.ShapeDtypeStruct((NC, n_src, ch), _F32),
                   jax.ShapeDtypeStruct((NC, n_dst, ch), _F32)],
    )(x_src, x_dst, time_src[:, None], seed_time[None, :],
      batch_src[:, None], emb_rows,
      W_enc_src, b_enc_src[None, :], W_enc_dst, b_enc_dst[None, :],
      W_time, b_time[None, :], id_emb)

    agg2 = _make_sc_agg(n_dst, c, e_rows, 2)
    p_sd0, p_ds0 = agg2(hs_sp, hd_sp, src_idx16, dst_idx16)
    p_sd0 = p_sd0.reshape(NC, -1, ch)
    p_ds0 = p_ds0.reshape(NC, -1, ch)

    h_dst1, hs1_sp = pl.pallas_call(
        functools.partial(_upd_body, ch),
        grid=grid,
        in_specs=[row(c), row(c), part_spec, part_spec, cnt_spec, cnt_spec,
                  full((c, c)), full((c, c)), full((c, c)), full((c, c))],
        out_specs=[row(c), split_spec],
        out_shape=[jax.ShapeDtypeStruct((n_dst, c), _F32),
                   jax.ShapeDtypeStruct((NC, n_src, ch), _F32)],
    )(h_dst, h_src, p_sd0, p_ds0, cnt_d, cnt_s,
      W_root_d0, W_nbr_d0, W_root_s0, W_nbr_s0)

    agg1 = _make_sc_agg(n_dst, c, e_rows, 1)
    (p_sd1,) = agg1(hs1_sp, src_idx16, dst_idx16)
    p_sd1 = p_sd1.reshape(NC, -1, ch)

    out = pl.pallas_call(
        _fin_body,
        grid=grid,
        in_specs=[row(c), part_spec, cnt_spec, full((c, c)), full((c, c)),
                  full((1, c)), full((1, c)), full((1, c)), full((1, 1))],
        out_specs=row(1),
        out_shape=jax.ShapeDtypeStruct((n_dst, 1), _F32),
    )(h_dst1, p_sd1, cnt_d, W_root_d1, W_nbr_d1,
      ln_gamma[None, :], ln_beta[None, :], W_head.T, b_head[:, None])

    return out.reshape(-1)


# speedup vs baseline: 8.5693x; 1.0211x over previous
import functools

import jax
import jax.numpy as jnp
from jax import lax
from jax.experimental import pallas as pl
from jax.experimental.pallas import tpu as pltpu
from jax.experimental.pallas import tpu_sc as plsc

_SC_PARAMS = pltpu.CompilerParams(use_tc_tiling_on_sc=False)

NC = 2
NS = 16
NW = NC * NS
LANE = 16
CHUNK = 125
NBUF = 5

_F32 = jnp.float32
_HIGH = lax.Precision.HIGHEST


def _fill2d(ref, rows, cols, value):
    vec = jnp.full((LANE,), value, _F32)

    def row_body(i, _):
        def col_body(j, __):
            ref[i, pl.ds(j * LANE, LANE)] = vec
            return 0
        return lax.fori_loop(0, cols // LANE, col_body, 0)

    lax.fori_loop(0, rows, row_body, 0)


def _padded_rows(n):
    return 128 * -(-n // (NS * 128))


@functools.lru_cache(maxsize=None)
def _make_sc_pre(vocab, c, nid_rows, e_rows, n_nodes):
    emb_rows_w = nid_rows // NW
    er_w = e_rows // NW
    zt = _padded_rows(n_nodes)
    npad = NS * zt
    mesh = plsc.VectorSubcoreMesh(core_axis_name="c", subcore_axis_name="s")

    @functools.partial(
        pl.kernel,
        out_type=[
            jax.ShapeDtypeStruct((nid_rows * 128, c), _F32),
            jax.ShapeDtypeStruct((NC, NS, zt, LANE), _F32),
            jax.ShapeDtypeStruct((NC, NS, zt, LANE), _F32),
        ],
        mesh=mesh,
        compiler_params=_SC_PARAMS,
        scratch_types=[
            pltpu.VMEM((emb_rows_w, 128), jnp.int32),
            pltpu.VMEM((128, c), _F32),
            pltpu.VMEM((128, c), _F32),
            pltpu.VMEM((er_w, CHUNK), jnp.int32),
            pltpu.VMEM((er_w, CHUNK), jnp.int32),
            pltpu.VMEM((CHUNK, LANE), _F32),
            pltpu.VMEM((zt, LANE), _F32),
            pltpu.VMEM_SHARED((npad, LANE), _F32),
            pltpu.VMEM_SHARED((npad, LANE), _F32),
            pltpu.SemaphoreType.DMA,
            pltpu.SemaphoreType.DMA,
            pltpu.SemaphoreType.DMA,
            pltpu.SemaphoreType.DMA,
        ],
    )
    def pre(emb_hbm, nid_hbm, src_hbm, dst_hbm,
            emb_out, cntd_out, cnts_out,
            nid_v, rows_a, rows_b, src_v, dst_v, ones_v, zero_v, accd, accs,
            sem_a, sem_b, sem_d, sem_s):
        cid = lax.axis_index("c")
        sid = lax.axis_index("s")
        w = cid * NS + sid

        _fill2d(ones_v, CHUNK, LANE, 1.0)
        _fill2d(zero_v, zt, LANE, 0.0)
        pltpu.sync_copy(src_hbm.at[w], src_v)
        pltpu.sync_copy(dst_hbm.at[w], dst_v)
        pltpu.sync_copy(zero_v, accd.at[pl.ds(sid * zt, zt), :])
        pltpu.sync_copy(zero_v, accs.at[pl.ds(sid * zt, zt), :])
        plsc.subcore_barrier()

        def cnt_body(i, _):
            pltpu.async_copy(ones_v, accd.at[dst_v.at[i]], sem_d, add=True)
            pltpu.async_copy(ones_v, accs.at[src_v.at[i]], sem_s, add=True)
            return 0

        lax.fori_loop(0, er_w, cnt_body, 0)

        pltpu.sync_copy(nid_hbm.at[w], nid_v)
        rings = [(rows_a, sem_a), (rows_b, sem_b)]
        for r in range(min(2, emb_rows_w)):
            buf, sem = rings[r % 2]
            pltpu.async_copy(emb_hbm.at[nid_v.at[r]], buf, sem)
        for r in range(emb_rows_w):
            buf, sem = rings[r % 2]
            pltpu.make_async_copy(emb_hbm.at[nid_v.at[0]], buf, sem).wait()
            pltpu.sync_copy(
                buf, emb_out.at[pl.ds((w * emb_rows_w + r) * 128, 128), :])
            if r + 2 < emb_rows_w:
                pltpu.async_copy(emb_hbm.at[nid_v.at[r + 2]], buf, sem)

        def cnt_drain(i, _):
            pltpu.make_async_copy(ones_v, accd.at[dst_v.at[0]], sem_d).wait()
            pltpu.make_async_copy(ones_v, accs.at[src_v.at[0]], sem_s).wait()
            return 0

        lax.fori_loop(0, er_w, cnt_drain, 0)
        plsc.subcore_barrier()
        pltpu.sync_copy(accd.at[pl.ds(sid * zt, zt), :], cntd_out.at[cid, sid])
        pltpu.sync_copy(accs.at[pl.ds(sid * zt, zt), :], cnts_out.at[cid, sid])

    return pre


@functools.lru_cache(maxsize=None)
def _make_sc_agg(n_nodes, c, e_rows, ndir):
    ch = c // NC
    er_w = e_rows // NS
    zt = _padded_rows(n_nodes)
    nz = zt // 128
    npad = NS * zt
    mesh = plsc.VectorSubcoreMesh(core_axis_name="c", subcore_axis_name="s")

    @functools.partial(
        pl.kernel,
        out_type=[jax.ShapeDtypeStruct((NC, NS * nz, 128, ch), _F32)
                  for _ in range(ndir)],
        mesh=mesh,
        compiler_params=_SC_PARAMS,
        scratch_types=[
            pltpu.VMEM((er_w, CHUNK), jnp.int32),
            pltpu.VMEM((er_w, CHUNK), jnp.int32),
        ] + [pltpu.VMEM((CHUNK, ch), _F32) for _ in range(NBUF)] + [
            pltpu.VMEM((128, ch), _F32),
            pltpu.VMEM_SHARED((npad, ch), _F32),
        ] + [pltpu.SemaphoreType.DMA for _ in range(NBUF)],
    )
    def agg(*refs):
        h_hbm = refs[0:ndir]
        src_hbm, dst_hbm = refs[ndir], refs[ndir + 1]
        outs = refs[ndir + 2: 2 * ndir + 2]
        rest = refs[2 * ndir + 2:]
        src_v, dst_v = rest[0], rest[1]
        rows = rest[2:2 + NBUF]
        zero_v, acc = rest[2 + NBUF], rest[3 + NBUF]
        sems = rest[4 + NBUF:]

        cid = lax.axis_index("c")
        sid = lax.axis_index("s")
        _fill2d(zero_v, 128, ch, 0.0)
        pltpu.sync_copy(src_hbm.at[sid], src_v)
        pltpu.sync_copy(dst_hbm.at[sid], dst_v)

        for d in range(ndir):
            gid_v = src_v if d == 0 else dst_v
            sid_v = dst_v if d == 0 else src_v
            for k in range(nz):
                pltpu.sync_copy(
                    zero_v, acc.at[pl.ds(sid * zt + k * 128, 128), :])
            plsc.subcore_barrier()

            h = h_hbm[d].at[cid]
            g, s = gid_v, sid_v

            def _wait(b, h=h, g=g):
                pltpu.make_async_copy(h.at[g.at[0]], rows[b], sems[b]).wait()

            for b in range(NBUF):
                pltpu.async_copy(h.at[g.at[b]], rows[b], sems[b])

            def ring_body(p, _, h=h, g=g, s=s):
                j = NBUF * p
                for b in range(NBUF):
                    _wait(b)
                    pltpu.sync_copy(rows[b], acc.at[s.at[j + b]], add=True)
                    pltpu.async_copy(h.at[g.at[j + b + NBUF]], rows[b],
                                     sems[b])
                return 0

            lax.fori_loop(0, er_w // NBUF - 1, ring_body, 0)
            for b in range(NBUF):
                _wait(b)
                pltpu.sync_copy(
                    rows[b], acc.at[s.at[er_w - NBUF + b]], add=True)
            plsc.subcore_barrier()
            for k in range(nz):
                sl = pl.ds(sid * zt + k * 128, 128)
                pltpu.sync_copy(acc.at[sl, :], outs[d].at[cid, sid * nz + k])
            if d + 1 < ndir:
                plsc.subcore_barrier()

    return agg


def _dot(a, b):
    return jnp.dot(a, b, preferred_element_type=_F32)


def _enc_body(nrows, blk, bsz, ch,
              xs_ref, xd_ref, ts_ref, st_ref, bsrc_ref, emb_ref,
              wes_ref, bes_ref, wed_ref, bed_ref,
              wt_ref, bt_ref, ide_ref,
              hs_out, hd_out, hss_out, hds_out):
    i = pl.program_id(0)
    hs = _dot(xs_ref[...], wes_ref[...]) + bes_ref[0][None, :]
    bs = bsrc_ref[...]
    cols = lax.broadcasted_iota(jnp.int32, (blk, bsz), 1)
    stg = jnp.sum(jnp.where(cols == bs, st_ref[...], 0.0), axis=1,
                  keepdims=True)
    rel = stg - ts_ref[...]
    hs = hs + rel * wt_ref[0][None, :] + bt_ref[0][None, :]
    row = i * blk + lax.broadcasted_iota(jnp.int32, (blk, 1), 0)
    hs = hs + jnp.where(row < bsz, 1.0, 0.0) * ide_ref[0][None, :]
    hs_out[...] = hs
    hd = (_dot(xd_ref[...], wed_ref[...]) + bed_ref[0][None, :]
          + emb_ref[...])
    hd_out[...] = hd
    hss_out[0] = hs[:, :ch]
    hss_out[1] = hs[:, ch:]
    hds_out[0] = hd[:, :ch]
    hds_out[1] = hd[:, ch:]


def _inv(cnt_ref):
    return (1.0 / jnp.maximum(cnt_ref[0, :, 0] + cnt_ref[1, :, 0],
                              1.0))[:, None]


def _upd_body(ch, hd_ref, hs_ref, psd_ref, pds_ref, cntd_ref, cnts_ref,
              wrd_ref, wnd_ref, wrs_ref, wns_ref, hd_out, hss_out):
    m_sd = jnp.concatenate([psd_ref[0], psd_ref[1]], axis=-1) * _inv(cntd_ref)
    m_ds = jnp.concatenate([pds_ref[0], pds_ref[1]], axis=-1) * _inv(cnts_ref)
    hd_out[...] = jnp.maximum(
        _dot(hd_ref[...], wrd_ref[...]) + _dot(m_sd, wnd_ref[...]), 0.0)
    hs1 = jnp.maximum(
        _dot(hs_ref[...], wrs_ref[...]) + _dot(m_ds, wns_ref[...]), 0.0)
    hss_out[0] = hs1[:, :ch]
    hss_out[1] = hs1[:, ch:]


def _fin_body(hd_ref, psd_ref, cntd_ref, wrd_ref, wnd_ref,
              gam_ref, bet_ref, wh_ref, bh_ref, out_ref):
    m_sd = jnp.concatenate([psd_ref[0], psd_ref[1]], axis=-1) * _inv(cntd_ref)
    h2 = jnp.maximum(
        _dot(hd_ref[...], wrd_ref[...]) + _dot(m_sd, wnd_ref[...]), 0.0)
    mu = jnp.mean(h2, axis=-1, keepdims=True)
    var = jnp.mean((h2 - mu) ** 2, axis=-1, keepdims=True)
    hn = (h2 - mu) / jnp.sqrt(var + 1e-5) * gam_ref[0][None, :] \
        + bet_ref[0][None, :]
    out_ref[...] = jnp.sum(hn * wh_ref[0][None, :], axis=1,
                           keepdims=True) + bh_ref[0, 0]


def _row_spec(blk, width):
    return pl.BlockSpec((blk, width), lambda i: (i, 0))


def _full_spec(shape):
    return pl.BlockSpec(shape, lambda i: tuple(0 for _ in shape))


def kernel(x_src, x_dst, time_src, seed_time, W_enc_src, b_enc_src,
           W_enc_dst, b_enc_dst, W_time, b_time, emb_shallow, id_emb,
           W_root_s0, W_nbr_s0, W_root_d0, W_nbr_d0, W_root_s1, W_nbr_s1,
           W_root_d1, W_nbr_d1, ln_gamma, ln_beta, W_head, b_head,
           batch_src, n_id_dst, edge_index):
    n_src, d_in = x_src.shape
    n_dst = x_dst.shape[0]
    c = W_enc_src.shape[1]
    e = edge_index.shape[1]
    bsz = seed_time.shape[0]
    vocab = emb_shallow.shape[0]
    assert e % (NW * CHUNK) == 0 and n_dst % (NS * 5) == 0 and n_src % (NS * 5) == 0

    e_rows = e // CHUNK
    src_idx = edge_index[0].reshape(NW, e_rows // NW, CHUNK)
    dst_idx = edge_index[1].reshape(NW, e_rows // NW, CHUNK)
    src_idx16 = edge_index[0].reshape(NS, e_rows // NS, CHUNK)
    dst_idx16 = edge_index[1].reshape(NS, e_rows // NS, CHUNK)
    ch = c // NC

    nid_pad = NW * 128 * -(-n_dst // (NW * 128))
    nid3d = jnp.concatenate(
        [n_id_dst, jnp.zeros((nid_pad - n_dst,), jnp.int32)]
    ).reshape(NW, -1, 128)

    assert n_src == n_dst
    sc_pre = _make_sc_pre(vocab, c, nid_pad // 128, e_rows, n_dst)
    emb_rows, cnt_d, cnt_s = sc_pre(emb_shallow, nid3d, src_idx, dst_idx)
    cnt_d = cnt_d.reshape(NC, -1, LANE)
    cnt_s = cnt_s.reshape(NC, -1, LANE)

    blk = 400
    grid = (n_src // blk,)
    row = functools.partial(_row_spec, blk)
    full = _full_spec
    cnt_spec = pl.BlockSpec((NC, blk, LANE), lambda i: (0, i, 0))
    part_spec = pl.BlockSpec((NC, blk, ch), lambda i: (0, i, 0))
    split_spec = pl.BlockSpec((NC, blk, ch), lambda i: (0, i, 0))

    h_src, h_dst, hs_sp, hd_sp = pl.pallas_call(
        functools.partial(_enc_body, n_src, blk, bsz, ch),
        grid=grid,
        in_specs=[row(d_in), row(d_in), row(1), full((1, bsz)), row(1),
                  row(c), full((d_in, c)), full((1, c)),
                  full((d_in, c)), full((1, c)), full((1, c)), full((1, c)),
                  full((1, c))],
        out_specs=[row(c), row(c), split_spec, split_spec],
        out_shape=[jax.ShapeDtypeStruct((n_src, c), _F32),
                   jax.ShapeDtypeStruct((n_dst, c), _F32),
                   jax.ShapeDtypeStruct((NC, n_src, ch), _F32),
                   jax.ShapeDtypeStruct((NC, n_dst, ch), _F32)],
    )(x_src, x_dst, time_src[:, None], seed_time[None, :],
      batch_src[:, None], emb_rows,
      W_enc_src, b_enc_src[None, :], W_enc_dst, b_enc_dst[None, :],
      W_time, b_time[None, :], id_emb)

    agg2 = _make_sc_agg(n_dst, c, e_rows, 2)
    p_sd0, p_ds0 = agg2(hs_sp, hd_sp, src_idx16, dst_idx16)
    p_sd0 = p_sd0.reshape(NC, -1, ch)
    p_ds0 = p_ds0.reshape(NC, -1, ch)

    h_dst1, hs1_sp = pl.pallas_call(
        functools.partial(_upd_body, ch),
        grid=grid,
        in_specs=[row(c), row(c), part_spec, part_spec, cnt_spec, cnt_spec,
                  full((c, c)), full((c, c)), full((c, c)), full((c, c))],
        out_specs=[row(c), split_spec],
        out_shape=[jax.ShapeDtypeStruct((n_dst, c), _F32),
                   jax.ShapeDtypeStruct((NC, n_src, ch), _F32)],
    )(h_dst, h_src, p_sd0, p_ds0, cnt_d, cnt_s,
      W_root_d0, W_nbr_d0, W_root_s0, W_nbr_s0)

    agg1 = _make_sc_agg(n_dst, c, e_rows, 1)
    (p_sd1,) = agg1(hs1_sp, src_idx16, dst_idx16)
    p_sd1 = p_sd1.reshape(NC, -1, ch)

    out = pl.pallas_call(
        _fin_body,
        grid=grid,
        in_specs=[row(c), part_spec, cnt_spec, full((c, c)), full((c, c)),
                  full((1, c)), full((1, c)), full((1, c)), full((1, 1))],
        out_specs=row(1),
        out_shape=jax.ShapeDtypeStruct((n_dst, 1), _F32),
    )(h_dst1, p_sd1, cnt_d, W_root_d1, W_nbr_d1,
      ln_gamma[None, :], ln_beta[None, :], W_head.T, b_head[:, None])

    return out.reshape(-1)
